# baseline (speedup 1.0000x reference)
import functools

import jax
import jax.numpy as jnp
from jax import lax
from jax.experimental import pallas as pl
from jax.experimental.pallas import tpu as pltpu
from jax.experimental.pallas import tpu_sc as plsc

TT = 2
NN = 10000
DD = 128
EE = 320000

NC = 2
NS = 16
NP = 10240
RPT = NP // NS

CH = 128
ECH = 64
NBUF = 4
BLK = 16
EPT = 20480
NCHUNK = EPT // CH
NBLK = NCHUNK // BLK
ENCHUNK = EPT // ECH
ENBLK = ENCHUNK // BLK

_MESH = plsc.VectorSubcoreMesh(
    core_axis_name="c", subcore_axis_name="s", num_cores=NC, num_subcores=NS
)


def _set_iidx(iidx_v, base, n):
    for m in range(n // 16):
        iidx_v[pl.ds(m * 16, 16)] = lax.iota(jnp.int32, 16) + base + m * 16


@functools.partial(
    pl.kernel,
    out_type=jax.ShapeDtypeStruct((TT, NP, DD), jnp.float32),
    mesh=_MESH,
    scratch_types=[
        pltpu.VMEM((BLK, CH), jnp.int32),
        pltpu.VMEM((CH, DD), jnp.float32),
        pltpu.VMEM((CH, DD), jnp.float32),
        pltpu.VMEM((CH,), jnp.int32),
        pltpu.VMEM_SHARED((NP, DD), jnp.float32),
    ],
)
def _deg_kernel(dst_hbm, deg_hbm, idx_v, ones_v, zb_v, iidx_v, acc_sh):
    c = lax.axis_index("c")
    s = lax.axis_index("s")

    def fill(i, _):
        for k in range(DD // 16):
            ones_v[i, pl.ds(k * 16, 16)] = jnp.ones((16,), jnp.float32)
            zb_v[i, pl.ds(k * 16, 16)] = jnp.zeros((16,), jnp.float32)
        return 0

    lax.fori_loop(0, CH, fill, 0)
    for k in range(RPT // CH):
        _set_iidx(iidx_v, s * RPT + k * CH, CH)
        pltpu.sync_copy(zb_v, acc_sh.at[iidx_v])
    plsc.subcore_barrier()

    def blk_body(b, _):
        pltpu.sync_copy(dst_hbm.at[c, s, pl.ds(b * BLK, BLK)], idx_v)
        for j in range(BLK):
            pltpu.sync_copy(ones_v, acc_sh.at[idx_v.at[j]], add=True)
        return 0

    lax.fori_loop(0, NBLK, blk_body, 0)
    plsc.subcore_barrier()
    for k in range(RPT // CH):
        _set_iidx(iidx_v, s * RPT + k * CH, CH)
        pltpu.sync_copy(acc_sh.at[iidx_v], zb_v)
        pltpu.sync_copy(zb_v, deg_hbm.at[c, pl.ds(s * RPT + k * CH, CH)])


@functools.partial(
    pl.kernel,
    out_type=jax.ShapeDtypeStruct((TT, NP, DD), jnp.float32),
    mesh=_MESH,
    scratch_types=[
        pltpu.VMEM((BLK, ECH), jnp.int32),
        pltpu.VMEM((BLK, ECH), jnp.int32),
        pltpu.VMEM((ECH, DD), jnp.float32),
        pltpu.VMEM((ECH, DD), jnp.float32),
        pltpu.VMEM((ECH, DD), jnp.float32),
        pltpu.VMEM((ECH, DD), jnp.float32),
        pltpu.VMEM((ECH,), jnp.int32),
        pltpu.VMEM_SHARED((NP, DD), jnp.float32),
        pltpu.SemaphoreType.DMA,
        pltpu.SemaphoreType.DMA,
        pltpu.SemaphoreType.DMA,
        pltpu.SemaphoreType.DMA,
    ],
)
def _edge_kernel(g_hbm, src_hbm, dst_hbm, es_hbm, sidx_v, didx_v,
                 rows_a, rows_b, rows_c, rows_d, iidx_v, acc_sh,
                 sem_a, sem_b, sem_c, sem_d):
    c = lax.axis_index("c")
    s = lax.axis_index("s")

    def fill0(i, _):
        for k in range(DD // 16):
            rows_a[i, pl.ds(k * 16, 16)] = jnp.zeros((16,), jnp.float32)
        return 0

    lax.fori_loop(0, ECH, fill0, 0)
    for k in range(RPT // ECH):
        _set_iidx(iidx_v, s * RPT + k * ECH, ECH)
        pltpu.sync_copy(rows_a, acc_sh.at[iidx_v])
    plsc.subcore_barrier()

    bufs = (rows_a, rows_b, rows_c, rows_d)
    sems = (sem_a, sem_b, sem_c, sem_d)
    DEPTH = NBUF - 1

    def blk_body(b, _):
        pltpu.sync_copy(src_hbm.at[c, s, pl.ds(b * BLK, BLK)], sidx_v)
        pltpu.sync_copy(dst_hbm.at[c, s, pl.ds(b * BLK, BLK)], didx_v)
        cps = [
            pltpu.async_copy(g_hbm.at[sidx_v.at[j]], bufs[j % NBUF], sems[j % NBUF])
            for j in range(DEPTH)
        ]
        for j in range(BLK):
            if j + DEPTH < BLK:
                cps.append(pltpu.async_copy(
                    g_hbm.at[sidx_v.at[j + DEPTH]],
                    bufs[(j + DEPTH) % NBUF],
                    sems[(j + DEPTH) % NBUF],
                ))
            cps[j].wait()
            pltpu.sync_copy(bufs[j % NBUF], acc_sh.at[didx_v.at[j]], add=True)
        return 0

    lax.fori_loop(0, ENBLK, blk_body, 0)
    plsc.subcore_barrier()
    for k in range(RPT // ECH):
        _set_iidx(iidx_v, s * RPT + k * ECH, ECH)
        pltpu.sync_copy(acc_sh.at[iidx_v], rows_a)
        pltpu.sync_copy(rows_a, es_hbm.at[c, pl.ds(s * RPT + k * ECH, ECH)])


_BN = 512


def _dinv_of(deg_ref):
    return lax.rsqrt(deg_ref[0, :, 0:1] + 1.0)


def _mm_scale_body(deg_ref, x_ref, w_ref, g_ref):
    dinv = _dinv_of(deg_ref)
    h = jnp.dot(x_ref[0], w_ref[0], preferred_element_type=jnp.float32)
    g_ref[0] = h * dinv


def _mid_body(deg_ref, es_ref, g_ref, w_ref, b_ref, g1_ref):
    dinv = _dinv_of(deg_ref)
    v = (es_ref[0] + g_ref[0]) * dinv + b_ref[0]
    y = jnp.where(v > 0, v, 0.2 * v)
    g1_ref[0] = jnp.dot(y, w_ref[0], preferred_element_type=jnp.float32) * dinv


def _final_body(deg_ref, es_ref, g_ref, b_ref, y_ref):
    dinv = _dinv_of(deg_ref)
    v = (es_ref[0] + g_ref[0]) * dinv + b_ref[0]
    y_ref[0] = jnp.where(v > 0, v, 0.2 * v)


def _node_spec(d):
    return pl.BlockSpec((1, _BN, d), lambda t, i: (t, i, 0))


def _w_spec():
    return pl.BlockSpec((1, DD, DD), lambda t, i: (t, 0, 0))


def _b_spec():
    return pl.BlockSpec((1, 1, DD), lambda t, i: (t, 0, 0))


_GRID = (TT, NP // _BN)
_OUT_TND = jax.ShapeDtypeStruct((TT, NP, DD), jnp.float32)

_mm_scale = pl.pallas_call(
    _mm_scale_body,
    grid=_GRID,
    in_specs=[_node_spec(DD), _node_spec(DD), _w_spec()],
    out_specs=_node_spec(DD),
    out_shape=_OUT_TND,
)

_mid = pl.pallas_call(
    _mid_body,
    grid=_GRID,
    in_specs=[_node_spec(DD), _node_spec(DD), _node_spec(DD), _w_spec(), _b_spec()],
    out_specs=_node_spec(DD),
    out_shape=_OUT_TND,
)

_final = pl.pallas_call(
    _final_body,
    grid=_GRID,
    in_specs=[_node_spec(DD), _node_spec(DD), _node_spec(DD), _b_spec()],
    out_specs=_node_spec(DD),
    out_shape=_OUT_TND,
)


@jax.jit
def kernel(x, edge_index, Ws, bs):
    src = edge_index[:, 0, :]
    dst = edge_index[:, 1, :]

    pad = EPT - EE // NS
    srcp = jnp.pad(src.reshape(TT, NS, EE // NS), ((0, 0), (0, 0), (0, pad)),
                   constant_values=NP - 1)
    dstp = jnp.pad(dst.reshape(TT, NS, EE // NS), ((0, 0), (0, 0), (0, pad)),
                   constant_values=NP - 1)
    srco = srcp + (jnp.arange(TT, dtype=jnp.int32) * NP)[:, None, None]
    srco = srco.reshape(TT, NS, ENCHUNK, ECH)
    dstq = dstp.reshape(TT, NS, ENCHUNK, ECH)
    dstp = dstp.reshape(TT, NS, NCHUNK, CH)

    xp = jnp.pad(x, ((0, 0), (0, NP - NN), (0, 0)))

    deg = _deg_kernel(dstp)

    Wa = Ws[0::2]
    Wb = Ws[1::2]
    ba = bs[0::2].reshape(TT, 1, DD)
    bb = bs[1::2].reshape(TT, 1, DD)

    g0 = _mm_scale(deg, xp, Wa)
    es0 = _edge_kernel(g0.reshape(TT * NP, DD), srco, dstq)
    g1 = _mid(deg, es0, g0, Wb, ba)
    es1 = _edge_kernel(g1.reshape(TT * NP, DD), srco, dstq)
    y = _final(deg, es1, g1, bb)
    return y[:, :NN, :]


# speedup vs baseline: 1.0267x; 1.0267x over previous
import functools

import jax
import jax.numpy as jnp
from jax import lax
from jax.experimental import pallas as pl
from jax.experimental.pallas import tpu as pltpu
from jax.experimental.pallas import tpu_sc as plsc

TT = 2
NN = 10000
DD = 128
EE = 320000

NC = 2
NS = 16
NP = 10240
RPT = NP // NS

CH = 128
ECH = 128
NBUF = 2
BLK = 32
EPT = 20480
NCHUNK = EPT // CH
NBLK = NCHUNK // BLK
ENCHUNK = EPT // ECH
ENBLK = ENCHUNK // BLK

_MESH = plsc.VectorSubcoreMesh(
    core_axis_name="c", subcore_axis_name="s", num_cores=NC, num_subcores=NS
)


def _set_iidx(iidx_v, base, n):
    for m in range(n // 16):
        iidx_v[pl.ds(m * 16, 16)] = lax.iota(jnp.int32, 16) + base + m * 16


@functools.partial(
    pl.kernel,
    out_type=jax.ShapeDtypeStruct((TT, NP, DD), jnp.float32),
    mesh=_MESH,
    scratch_types=[
        pltpu.VMEM((BLK, CH), jnp.int32),
        pltpu.VMEM((CH, DD), jnp.float32),
        pltpu.VMEM((CH, DD), jnp.float32),
        pltpu.VMEM((CH,), jnp.int32),
        pltpu.VMEM_SHARED((NP, DD), jnp.float32),
    ],
)
def _deg_kernel(dst_hbm, deg_hbm, idx_v, ones_v, zb_v, iidx_v, acc_sh):
    c = lax.axis_index("c")
    s = lax.axis_index("s")

    def fill(i, _):
        for k in range(DD // 16):
            ones_v[i, pl.ds(k * 16, 16)] = jnp.ones((16,), jnp.float32)
            zb_v[i, pl.ds(k * 16, 16)] = jnp.zeros((16,), jnp.float32)
        return 0

    lax.fori_loop(0, CH, fill, 0)
    for k in range(RPT // CH):
        _set_iidx(iidx_v, s * RPT + k * CH, CH)
        pltpu.sync_copy(zb_v, acc_sh.at[iidx_v])
    plsc.subcore_barrier()

    def blk_body(b, _):
        pltpu.sync_copy(dst_hbm.at[c, s, pl.ds(b * BLK, BLK)], idx_v)
        for j in range(BLK):
            pltpu.sync_copy(ones_v, acc_sh.at[idx_v.at[j]], add=True)
        return 0

    lax.fori_loop(0, NBLK, blk_body, 0)
    plsc.subcore_barrier()
    for k in range(RPT // CH):
        _set_iidx(iidx_v, s * RPT + k * CH, CH)
        pltpu.sync_copy(acc_sh.at[iidx_v], zb_v)
        pltpu.sync_copy(zb_v, deg_hbm.at[c, pl.ds(s * RPT + k * CH, CH)])


@functools.partial(
    pl.kernel,
    out_type=jax.ShapeDtypeStruct((TT, NP, DD), jnp.float32),
    mesh=_MESH,
    scratch_types=[
        pltpu.VMEM((BLK, ECH), jnp.int32),
        pltpu.VMEM((BLK, ECH), jnp.int32),
        pltpu.VMEM((ECH, DD), jnp.float32),
        pltpu.VMEM((ECH, DD), jnp.float32),
        pltpu.VMEM((ECH,), jnp.int32),
        pltpu.VMEM_SHARED((NP, DD), jnp.float32),
        pltpu.SemaphoreType.DMA,
        pltpu.SemaphoreType.DMA,
    ],
)
def _edge_kernel(g_hbm, src_hbm, dst_hbm, es_hbm, sidx_v, didx_v,
                 rows_a, rows_b, iidx_v, acc_sh, sem_a, sem_b):
    c = lax.axis_index("c")
    s = lax.axis_index("s")

    def fill0(i, _):
        for k in range(DD // 16):
            rows_a[i, pl.ds(k * 16, 16)] = jnp.zeros((16,), jnp.float32)
        return 0

    lax.fori_loop(0, ECH, fill0, 0)
    for k in range(RPT // ECH):
        _set_iidx(iidx_v, s * RPT + k * ECH, ECH)
        pltpu.sync_copy(rows_a, acc_sh.at[iidx_v])
    plsc.subcore_barrier()

    bufs = (rows_a, rows_b)
    sems = (sem_a, sem_b)
    DEPTH = NBUF - 1

    def blk_body(b, _):
        pltpu.sync_copy(src_hbm.at[c, s, pl.ds(b * BLK, BLK)], sidx_v)
        pltpu.sync_copy(dst_hbm.at[c, s, pl.ds(b * BLK, BLK)], didx_v)
        cps = [
            pltpu.async_copy(g_hbm.at[sidx_v.at[j]], bufs[j % NBUF], sems[j % NBUF])
            for j in range(DEPTH)
        ]
        for j in range(BLK):
            if j + DEPTH < BLK:
                cps.append(pltpu.async_copy(
                    g_hbm.at[sidx_v.at[j + DEPTH]],
                    bufs[(j + DEPTH) % NBUF],
                    sems[(j + DEPTH) % NBUF],
                ))
            cps[j].wait()
            pltpu.sync_copy(bufs[j % NBUF], acc_sh.at[didx_v.at[j]], add=True)
        return 0

    lax.fori_loop(0, ENBLK, blk_body, 0)
    plsc.subcore_barrier()
    for k in range(RPT // ECH):
        _set_iidx(iidx_v, s * RPT + k * ECH, ECH)
        pltpu.sync_copy(acc_sh.at[iidx_v], rows_a)
        pltpu.sync_copy(rows_a, es_hbm.at[c, pl.ds(s * RPT + k * ECH, ECH)])


_BN = 512


def _dinv_of(deg_ref):
    return lax.rsqrt(deg_ref[0, :, 0:1] + 1.0)


def _mm_scale_body(deg_ref, x_ref, w_ref, g_ref):
    dinv = _dinv_of(deg_ref)
    h = jnp.dot(x_ref[0], w_ref[0], preferred_element_type=jnp.float32)
    g_ref[0] = h * dinv


def _mid_body(deg_ref, es_ref, g_ref, w_ref, b_ref, g1_ref):
    dinv = _dinv_of(deg_ref)
    v = (es_ref[0] + g_ref[0]) * dinv + b_ref[0]
    y = jnp.where(v > 0, v, 0.2 * v)
    g1_ref[0] = jnp.dot(y, w_ref[0], preferred_element_type=jnp.float32) * dinv


def _final_body(deg_ref, es_ref, g_ref, b_ref, y_ref):
    dinv = _dinv_of(deg_ref)
    v = (es_ref[0] + g_ref[0]) * dinv + b_ref[0]
    y_ref[0] = jnp.where(v > 0, v, 0.2 * v)


def _node_spec(d):
    return pl.BlockSpec((1, _BN, d), lambda t, i: (t, i, 0))


def _w_spec():
    return pl.BlockSpec((1, DD, DD), lambda t, i: (t, 0, 0))


def _b_spec():
    return pl.BlockSpec((1, 1, DD), lambda t, i: (t, 0, 0))


_GRID = (TT, NP // _BN)
_OUT_TND = jax.ShapeDtypeStruct((TT, NP, DD), jnp.float32)

_mm_scale = pl.pallas_call(
    _mm_scale_body,
    grid=_GRID,
    in_specs=[_node_spec(DD), _node_spec(DD), _w_spec()],
    out_specs=_node_spec(DD),
    out_shape=_OUT_TND,
)

_mid = pl.pallas_call(
    _mid_body,
    grid=_GRID,
    in_specs=[_node_spec(DD), _node_spec(DD), _node_spec(DD), _w_spec(), _b_spec()],
    out_specs=_node_spec(DD),
    out_shape=_OUT_TND,
)

_final = pl.pallas_call(
    _final_body,
    grid=_GRID,
    in_specs=[_node_spec(DD), _node_spec(DD), _node_spec(DD), _b_spec()],
    out_specs=_node_spec(DD),
    out_shape=_OUT_TND,
)


@jax.jit
def kernel(x, edge_index, Ws, bs):
    src = edge_index[:, 0, :]
    dst = edge_index[:, 1, :]

    pad = EPT - EE // NS
    srcp = jnp.pad(src.reshape(TT, NS, EE // NS), ((0, 0), (0, 0), (0, pad)),
                   constant_values=NP - 1)
    dstp = jnp.pad(dst.reshape(TT, NS, EE // NS), ((0, 0), (0, 0), (0, pad)),
                   constant_values=NP - 1)
    srco = srcp + (jnp.arange(TT, dtype=jnp.int32) * NP)[:, None, None]
    srco = srco.reshape(TT, NS, ENCHUNK, ECH)
    dstq = dstp.reshape(TT, NS, ENCHUNK, ECH)
    dstp = dstp.reshape(TT, NS, NCHUNK, CH)

    xp = jnp.pad(x, ((0, 0), (0, NP - NN), (0, 0)))

    deg = _deg_kernel(dstp)

    Wa = Ws[0::2]
    Wb = Ws[1::2]
    ba = bs[0::2].reshape(TT, 1, DD)
    bb = bs[1::2].reshape(TT, 1, DD)

    g0 = _mm_scale(deg, xp, Wa)
    es0 = _edge_kernel(g0.reshape(TT * NP, DD), srco, dstq)
    g1 = _mid(deg, es0, g0, Wb, ba)
    es1 = _edge_kernel(g1.reshape(TT * NP, DD), srco, dstq)
    y = _final(deg, es1, g1, bb)
    return y[:, :NN, :]


# speedup vs baseline: 1.0316x; 1.0049x over previous
import functools

import jax
import jax.numpy as jnp
from jax import lax
from jax.experimental import pallas as pl
from jax.experimental.pallas import tpu as pltpu
from jax.experimental.pallas import tpu_sc as plsc

TT = 2
NN = 10000
DD = 128
EE = 320000

NC = 2
NS = 16
NP = 10240
RPT = NP // NS

CH = 128
ECH = 128
NBUF = 2
BLK = 32
EPT = 20480
NCHUNK = EPT // CH
NBLK = NCHUNK // BLK
ENCHUNK = EPT // ECH
ENBLK = ENCHUNK // BLK

_MESH = plsc.VectorSubcoreMesh(
    core_axis_name="c", subcore_axis_name="s", num_cores=NC, num_subcores=NS
)


def _set_iidx(iidx_v, base, n):
    for m in range(n // 16):
        iidx_v[pl.ds(m * 16, 16)] = lax.iota(jnp.int32, 16) + base + m * 16


@functools.partial(
    pl.kernel,
    out_type=jax.ShapeDtypeStruct((TT, NP, DD), jnp.float32),
    mesh=_MESH,
    scratch_types=[
        pltpu.VMEM((BLK, CH), jnp.int32),
        pltpu.VMEM((CH, DD), jnp.float32),
        pltpu.VMEM((CH, DD), jnp.float32),
        pltpu.VMEM((CH,), jnp.int32),
        pltpu.VMEM_SHARED((NP, DD), jnp.float32),
        pltpu.SemaphoreType.DMA,
        pltpu.SemaphoreType.DMA,
    ],
)
def _deg_kernel(dst_hbm, deg_hbm, idx_v, ones_v, zb_v, iidx_v, acc_sh, dsem_a, dsem_b):
    c = lax.axis_index("c")
    s = lax.axis_index("s")

    def fill(i, _):
        for k in range(DD // 16):
            ones_v[i, pl.ds(k * 16, 16)] = jnp.ones((16,), jnp.float32)
            zb_v[i, pl.ds(k * 16, 16)] = jnp.zeros((16,), jnp.float32)
        return 0

    lax.fori_loop(0, CH, fill, 0)
    for k in range(RPT // CH):
        _set_iidx(iidx_v, s * RPT + k * CH, CH)
        pltpu.sync_copy(zb_v, acc_sh.at[iidx_v])
    plsc.subcore_barrier()

    def blk_body(b, _):
        pltpu.sync_copy(dst_hbm.at[c, s, pl.ds(b * BLK, BLK)], idx_v)
        for j in range(BLK):
            pltpu.sync_copy(ones_v, acc_sh.at[idx_v.at[j]], add=True)
        return 0

    lax.fori_loop(0, NBLK, blk_body, 0)
    plsc.subcore_barrier()
    dbufs = (zb_v, ones_v)
    dsems = (dsem_a, dsem_b)
    wr = [None, None]
    for k in range(RPT // CH):
        if wr[k % 2] is not None:
            wr[k % 2].wait()
        _set_iidx(iidx_v, s * RPT + k * CH, CH)
        pltpu.sync_copy(acc_sh.at[iidx_v], dbufs[k % 2])
        wr[k % 2] = pltpu.async_copy(
            dbufs[k % 2], deg_hbm.at[c, pl.ds(s * RPT + k * CH, CH)], dsems[k % 2]
        )
    for w in wr:
        if w is not None:
            w.wait()


@functools.partial(
    pl.kernel,
    out_type=jax.ShapeDtypeStruct((TT, NP, DD), jnp.float32),
    mesh=_MESH,
    scratch_types=[
        pltpu.VMEM((BLK, ECH), jnp.int32),
        pltpu.VMEM((BLK, ECH), jnp.int32),
        pltpu.VMEM((ECH, DD), jnp.float32),
        pltpu.VMEM((ECH, DD), jnp.float32),
        pltpu.VMEM((ECH,), jnp.int32),
        pltpu.VMEM_SHARED((NP, DD), jnp.float32),
        pltpu.SemaphoreType.DMA,
        pltpu.SemaphoreType.DMA,
    ],
)
def _edge_kernel(g_hbm, src_hbm, dst_hbm, es_hbm, sidx_v, didx_v,
                 rows_a, rows_b, iidx_v, acc_sh, sem_a, sem_b):
    c = lax.axis_index("c")
    s = lax.axis_index("s")

    def fill0(i, _):
        for k in range(DD // 16):
            rows_a[i, pl.ds(k * 16, 16)] = jnp.zeros((16,), jnp.float32)
        return 0

    lax.fori_loop(0, ECH, fill0, 0)
    for k in range(RPT // ECH):
        _set_iidx(iidx_v, s * RPT + k * ECH, ECH)
        pltpu.sync_copy(rows_a, acc_sh.at[iidx_v])
    plsc.subcore_barrier()

    bufs = (rows_a, rows_b)
    sems = (sem_a, sem_b)
    DEPTH = NBUF - 1

    def blk_body(b, _):
        pltpu.sync_copy(src_hbm.at[c, s, pl.ds(b * BLK, BLK)], sidx_v)
        pltpu.sync_copy(dst_hbm.at[c, s, pl.ds(b * BLK, BLK)], didx_v)
        cps = [
            pltpu.async_copy(g_hbm.at[sidx_v.at[j]], bufs[j % NBUF], sems[j % NBUF])
            for j in range(DEPTH)
        ]
        for j in range(BLK):
            if j + DEPTH < BLK:
                cps.append(pltpu.async_copy(
                    g_hbm.at[sidx_v.at[j + DEPTH]],
                    bufs[(j + DEPTH) % NBUF],
                    sems[(j + DEPTH) % NBUF],
                ))
            cps[j].wait()
            pltpu.sync_copy(bufs[j % NBUF], acc_sh.at[didx_v.at[j]], add=True)
        return 0

    lax.fori_loop(0, ENBLK, blk_body, 0)
    plsc.subcore_barrier()
    wr = [None, None]
    for k in range(RPT // ECH):
        if wr[k % 2] is not None:
            wr[k % 2].wait()
        _set_iidx(iidx_v, s * RPT + k * ECH, ECH)
        pltpu.sync_copy(acc_sh.at[iidx_v], bufs[k % 2])
        wr[k % 2] = pltpu.async_copy(
            bufs[k % 2], es_hbm.at[c, pl.ds(s * RPT + k * ECH, ECH)], sems[k % 2]
        )
    for w in wr:
        if w is not None:
            w.wait()


_BN = 512


def _dinv_of(deg_ref):
    return lax.rsqrt(deg_ref[0, :, 0:1] + 1.0)


def _mm_scale_body(deg_ref, x_ref, w_ref, g_ref):
    dinv = _dinv_of(deg_ref)
    h = jnp.dot(x_ref[0], w_ref[0], preferred_element_type=jnp.float32)
    g_ref[0] = h * dinv


def _mid_body(deg_ref, es_ref, g_ref, w_ref, b_ref, g1_ref):
    dinv = _dinv_of(deg_ref)
    v = (es_ref[0] + g_ref[0]) * dinv + b_ref[0]
    y = jnp.where(v > 0, v, 0.2 * v)
    g1_ref[0] = jnp.dot(y, w_ref[0], preferred_element_type=jnp.float32) * dinv


def _final_body(deg_ref, es_ref, g_ref, b_ref, y_ref):
    dinv = _dinv_of(deg_ref)
    v = (es_ref[0] + g_ref[0]) * dinv + b_ref[0]
    y_ref[0] = jnp.where(v > 0, v, 0.2 * v)


def _node_spec(d):
    return pl.BlockSpec((1, _BN, d), lambda t, i: (t, i, 0))


def _w_spec():
    return pl.BlockSpec((1, DD, DD), lambda t, i: (t, 0, 0))


def _b_spec():
    return pl.BlockSpec((1, 1, DD), lambda t, i: (t, 0, 0))


_GRID = (TT, NP // _BN)
_OUT_TND = jax.ShapeDtypeStruct((TT, NP, DD), jnp.float32)

_mm_scale = pl.pallas_call(
    _mm_scale_body,
    grid=_GRID,
    in_specs=[_node_spec(DD), _node_spec(DD), _w_spec()],
    out_specs=_node_spec(DD),
    out_shape=_OUT_TND,
)

_mid = pl.pallas_call(
    _mid_body,
    grid=_GRID,
    in_specs=[_node_spec(DD), _node_spec(DD), _node_spec(DD), _w_spec(), _b_spec()],
    out_specs=_node_spec(DD),
    out_shape=_OUT_TND,
)

_final = pl.pallas_call(
    _final_body,
    grid=_GRID,
    in_specs=[_node_spec(DD), _node_spec(DD), _node_spec(DD), _b_spec()],
    out_specs=_node_spec(DD),
    out_shape=_OUT_TND,
)


@jax.jit
def kernel(x, edge_index, Ws, bs):
    src = edge_index[:, 0, :]
    dst = edge_index[:, 1, :]

    pad = EPT - EE // NS
    srcp = jnp.pad(src.reshape(TT, NS, EE // NS), ((0, 0), (0, 0), (0, pad)),
                   constant_values=NP - 1)
    dstp = jnp.pad(dst.reshape(TT, NS, EE // NS), ((0, 0), (0, 0), (0, pad)),
                   constant_values=NP - 1)
    srco = srcp + (jnp.arange(TT, dtype=jnp.int32) * NP)[:, None, None]
    srco = srco.reshape(TT, NS, ENCHUNK, ECH)
    dstq = dstp.reshape(TT, NS, ENCHUNK, ECH)
    dstp = dstp.reshape(TT, NS, NCHUNK, CH)

    xp = jnp.pad(x, ((0, 0), (0, NP - NN), (0, 0)))

    deg = _deg_kernel(dstp)

    Wa = Ws[0::2]
    Wb = Ws[1::2]
    ba = bs[0::2].reshape(TT, 1, DD)
    bb = bs[1::2].reshape(TT, 1, DD)

    g0 = _mm_scale(deg, xp, Wa)
    es0 = _edge_kernel(g0.reshape(TT * NP, DD), srco, dstq)
    g1 = _mid(deg, es0, g0, Wb, ba)
    es1 = _edge_kernel(g1.reshape(TT * NP, DD), srco, dstq)
    y = _final(deg, es1, g1, bb)
    return y[:, :NN, :]
